# Initial kernel scaffold; baseline (speedup 1.0000x reference)
import functools

import jax
import jax.numpy as jnp
from jax import lax
from jax.experimental import pallas as pl
from jax.experimental.pallas import tpu as pltpu
from jax.experimental.pallas import tpu_sc as plsc

N = 10000
E = 320000
ROWS = 256
NBLK = (N + ROWS - 1) // ROWS
NC, NS = 2, 16
NW = NC * NS
EPW = E // NW
K = 80
NCHUNK = EPW // K
RPS = N // NS
FW = 144

_f32 = jnp.float32


def _prep_math(x, w, att_s, att_d, heads):
    h = jnp.dot(x, w, preferred_element_type=_f32)
    ch = 128 // heads
    a_s = (h * att_s).reshape(ROWS, heads, ch).sum(-1)
    a_d = (h * att_d).reshape(ROWS, heads, ch).sum(-1)
    pad = jnp.zeros((ROWS, 16 - heads), _f32)
    g = jnp.concatenate([h, a_s, pad], axis=1)
    ad = jnp.concatenate([a_d, pad], axis=1)
    return g, ad


def _tc_prep_body(heads, x_ref, w_ref, as_ref, ad_ref, g_ref, adout_ref):
    g, ad = _prep_math(x_ref[...], w_ref[...], as_ref[...], ad_ref[...], heads)
    g_ref[...] = g
    adout_ref[...] = ad


def _tc_prep(x, w, att_s, att_d, heads):
    return pl.pallas_call(
        functools.partial(_tc_prep_body, heads),
        grid=(NBLK,),
        in_specs=[
            pl.BlockSpec((ROWS, 128), lambda i: (i, 0)),
            pl.BlockSpec((128, 128), lambda i: (0, 0)),
            pl.BlockSpec((1, 128), lambda i: (0, 0)),
            pl.BlockSpec((1, 128), lambda i: (0, 0)),
        ],
        out_specs=[
            pl.BlockSpec((ROWS, FW), lambda i: (i, 0)),
            pl.BlockSpec((ROWS, 16), lambda i: (i, 0)),
        ],
        out_shape=[
            jax.ShapeDtypeStruct((N, FW), _f32),
            jax.ShapeDtypeStruct((N, 16), _f32),
        ],
    )(x, w, att_s, att_d)


def _combine_math(u0, u1, b, heads):
    u = u0 + u1
    s = u[:, 128:128 + heads]
    o = u[:, :128].reshape(ROWS, heads, 128 // heads)
    o = o / (s + 1e-16)[:, :, None]
    return o.reshape(ROWS, 128) + b


def _tc_mid_body(u0_ref, u1_ref, b_ref, w_ref, as_ref, ad_ref, g_ref, adout_ref):
    x1 = _combine_math(u0_ref[...], u1_ref[...], b_ref[...], 8)
    x1 = jnp.where(x1 > 0.0, x1, jnp.exp(x1) - 1.0)
    g, ad = _prep_math(x1, w_ref[...], as_ref[...], ad_ref[...], 1)
    g_ref[...] = g
    adout_ref[...] = ad


def _tc_mid(u0, u1, b, w, att_s, att_d):
    return pl.pallas_call(
        _tc_mid_body,
        grid=(NBLK,),
        in_specs=[
            pl.BlockSpec((ROWS, FW), lambda i: (i, 0)),
            pl.BlockSpec((ROWS, FW), lambda i: (i, 0)),
            pl.BlockSpec((1, 128), lambda i: (0, 0)),
            pl.BlockSpec((128, 128), lambda i: (0, 0)),
            pl.BlockSpec((1, 128), lambda i: (0, 0)),
            pl.BlockSpec((1, 128), lambda i: (0, 0)),
        ],
        out_specs=[
            pl.BlockSpec((ROWS, FW), lambda i: (i, 0)),
            pl.BlockSpec((ROWS, 16), lambda i: (i, 0)),
        ],
        out_shape=[
            jax.ShapeDtypeStruct((N, FW), _f32),
            jax.ShapeDtypeStruct((N, 16), _f32),
        ],
    )(u0, u1, b, w, att_s, att_d)


def _tc_final_body(u0_ref, u1_ref, b_ref, out_ref):
    out_ref[...] = _combine_math(u0_ref[...], u1_ref[...], b_ref[...], 1)


def _tc_final(u0, u1, b):
    return pl.pallas_call(
        _tc_final_body,
        grid=(NBLK,),
        in_specs=[
            pl.BlockSpec((ROWS, FW), lambda i: (i, 0)),
            pl.BlockSpec((ROWS, FW), lambda i: (i, 0)),
            pl.BlockSpec((1, 128), lambda i: (0, 0)),
        ],
        out_specs=pl.BlockSpec((ROWS, 128), lambda i: (i, 0)),
        out_shape=jax.ShapeDtypeStruct((N, 128), _f32),
    )(u0, u1, b)


def _make_sc_edge(lane_for_block):
    mesh = plsc.VectorSubcoreMesh(core_axis_name="c", subcore_axis_name="s")

    @functools.partial(
        pl.kernel,
        out_type=jax.ShapeDtypeStruct((NC, N, FW), _f32),
        mesh=mesh,
        scratch_types=[
            pltpu.VMEM_SHARED((N, FW), _f32),
            pltpu.VMEM((K, FW), _f32),
            pltpu.VMEM((K, 16), _f32),
            pltpu.VMEM((K,), jnp.int32),
            pltpu.VMEM((K,), jnp.int32),
            pltpu.SemaphoreType.DMA,
        ],
    )
    def sc_edge(g_hbm, ad_hbm, ei_hbm, z_hbm, u_hbm, u_sp, rows, adb, sidx,
                didx, sem):
        c = lax.axis_index("c")
        s = lax.axis_index("s")
        wid = c * NS + s
        pltpu.sync_copy(z_hbm.at[pl.ds(s * RPS, RPS)],
                        u_sp.at[pl.ds(s * RPS, RPS)])
        plsc.subcore_barrier()

        ebase = wid * EPW

        def chunk(ci, carry):
            eb = ebase + ci * K
            pltpu.sync_copy(ei_hbm.at[0, pl.ds(eb, K)], sidx)
            pltpu.sync_copy(ei_hbm.at[1, pl.ds(eb, K)], didx)
            pltpu.async_copy(g_hbm.at[sidx], rows, sem).wait()
            pltpu.async_copy(ad_hbm.at[didx], adb, sem).wait()

            def edge(j, carry2):
                z = rows[j, pl.ds(128, 16)] + adb[j]
                z = jnp.where(z > 0.0, z, 0.2 * z)
                p = jnp.exp(z)
                rows[j, pl.ds(128, 16)] = p
                for b in range(8):
                    pv = jnp.take(p,
                                  jnp.full((16,), lane_for_block[b], jnp.int32),
                                  mode="promise_in_bounds")
                    rows[j, pl.ds(b * 16, 16)] = rows[j, pl.ds(b * 16, 16)] * pv
                return carry2

            lax.fori_loop(0, K, edge, 0)
            pltpu.sync_copy(rows, u_sp.at[didx], add=True)
            return carry

        lax.fori_loop(0, NCHUNK, chunk, 0)
        plsc.subcore_barrier()
        pltpu.sync_copy(u_sp.at[pl.ds(s * RPS, RPS)],
                        u_hbm.at[c, pl.ds(s * RPS, RPS)])

    return sc_edge


_sc_edge_l0 = _make_sc_edge(tuple(range(8)))
_sc_edge_l1 = _make_sc_edge((0,) * 8)


def kernel(x, edge_attr, edge_index, W0, att_src0, att_dst0, b0,
           W1, att_src1, att_dst1, b1):
    del edge_attr
    as0 = att_src0.reshape(1, 128)
    ad0 = att_dst0.reshape(1, 128)
    as1 = att_src1.reshape(1, 128)
    ad1 = att_dst1.reshape(1, 128)
    b0f = b0.reshape(1, 128)
    b1f = b1.reshape(1, 128)
    zeros = jnp.zeros((N, FW), _f32)

    g0, adr0 = _tc_prep(x, W0, as0, ad0, 8)
    u0 = _sc_edge_l0(g0, adr0, edge_index, zeros)
    g1, adr1 = _tc_mid(u0[0], u0[1], b0f, W1, as1, ad1)
    u1 = _sc_edge_l1(g1, adr1, edge_index, zeros)
    return _tc_final(u1[0], u1[1], b1f)


# speedup vs baseline: 34.0032x; 34.0032x over previous
import functools

import jax
import jax.numpy as jnp
from jax import lax
from jax.experimental import pallas as pl
from jax.experimental.pallas import tpu as pltpu
from jax.experimental.pallas import tpu_sc as plsc

N = 10000
E = 320000
ROWS = 256
NBLK = (N + ROWS - 1) // ROWS
NC, NS = 2, 16
NW = NC * NS
EPW = E // NW
K = 80
NCHUNK = EPW // K
NPAD = 10240
RPS = NPAD // NS
FW = 144

_f32 = jnp.float32


def _prep_math(x, w, att_s, att_d, heads):
    h = jnp.dot(x, w, preferred_element_type=_f32)
    ch = 128 // heads
    a_s = (h * att_s).reshape(ROWS, heads, ch).sum(-1)
    a_d = (h * att_d).reshape(ROWS, heads, ch).sum(-1)
    pad = jnp.zeros((ROWS, 16 - heads), _f32)
    g = jnp.concatenate([h, a_s, pad], axis=1)
    ad = jnp.concatenate([a_d, pad], axis=1)
    return g, ad


def _tc_prep_body(heads, x_ref, w_ref, as_ref, ad_ref, g_ref, adout_ref):
    g, ad = _prep_math(x_ref[...], w_ref[...], as_ref[...], ad_ref[...], heads)
    g_ref[...] = g
    adout_ref[...] = ad


def _tc_prep(x, w, att_s, att_d, heads):
    return pl.pallas_call(
        functools.partial(_tc_prep_body, heads),
        grid=(NBLK,),
        in_specs=[
            pl.BlockSpec((ROWS, 128), lambda i: (i, 0)),
            pl.BlockSpec((128, 128), lambda i: (0, 0)),
            pl.BlockSpec((1, 128), lambda i: (0, 0)),
            pl.BlockSpec((1, 128), lambda i: (0, 0)),
        ],
        out_specs=[
            pl.BlockSpec((ROWS, FW), lambda i: (i, 0)),
            pl.BlockSpec((ROWS, 16), lambda i: (i, 0)),
        ],
        out_shape=[
            jax.ShapeDtypeStruct((NPAD, FW), _f32),
            jax.ShapeDtypeStruct((NPAD, 16), _f32),
        ],
    )(x, w, att_s, att_d)


def _combine_math(u0, u1, b, heads):
    u = u0 + u1
    s = u[:, 128:128 + heads]
    o = u[:, :128].reshape(ROWS, heads, 128 // heads)
    o = o / (s + 1e-16)[:, :, None]
    return o.reshape(ROWS, 128) + b


def _tc_mid_body(u0_ref, u1_ref, b_ref, w_ref, as_ref, ad_ref, g_ref, adout_ref):
    x1 = _combine_math(u0_ref[...], u1_ref[...], b_ref[...], 8)
    x1 = jnp.where(x1 > 0.0, x1, jnp.exp(x1) - 1.0)
    g, ad = _prep_math(x1, w_ref[...], as_ref[...], ad_ref[...], 1)
    g_ref[...] = g
    adout_ref[...] = ad


def _tc_mid(u0, u1, b, w, att_s, att_d):
    return pl.pallas_call(
        _tc_mid_body,
        grid=(NBLK,),
        in_specs=[
            pl.BlockSpec((ROWS, FW), lambda i: (i, 0)),
            pl.BlockSpec((ROWS, FW), lambda i: (i, 0)),
            pl.BlockSpec((1, 128), lambda i: (0, 0)),
            pl.BlockSpec((128, 128), lambda i: (0, 0)),
            pl.BlockSpec((1, 128), lambda i: (0, 0)),
            pl.BlockSpec((1, 128), lambda i: (0, 0)),
        ],
        out_specs=[
            pl.BlockSpec((ROWS, FW), lambda i: (i, 0)),
            pl.BlockSpec((ROWS, 16), lambda i: (i, 0)),
        ],
        out_shape=[
            jax.ShapeDtypeStruct((NPAD, FW), _f32),
            jax.ShapeDtypeStruct((NPAD, 16), _f32),
        ],
    )(u0, u1, b, w, att_s, att_d)


def _tc_final_body(u0_ref, u1_ref, b_ref, out_ref):
    out_ref[...] = _combine_math(u0_ref[...], u1_ref[...], b_ref[...], 1)


def _tc_final(u0, u1, b):
    return pl.pallas_call(
        _tc_final_body,
        grid=(NBLK,),
        in_specs=[
            pl.BlockSpec((ROWS, FW), lambda i: (i, 0)),
            pl.BlockSpec((ROWS, FW), lambda i: (i, 0)),
            pl.BlockSpec((1, 128), lambda i: (0, 0)),
        ],
        out_specs=pl.BlockSpec((ROWS, 128), lambda i: (i, 0)),
        out_shape=jax.ShapeDtypeStruct((N, 128), _f32),
    )(u0, u1, b)


def _make_sc_edge(lane_for_block):
    mesh = plsc.VectorSubcoreMesh(core_axis_name="c", subcore_axis_name="s")

    @functools.partial(
        pl.kernel,
        out_type=jax.ShapeDtypeStruct((NC, NPAD, FW), _f32),
        mesh=mesh,
        scratch_types=[
            pltpu.VMEM_SHARED((NPAD, FW), _f32),
            pltpu.VMEM((K, FW), _f32),
            pltpu.VMEM((K, 16), _f32),
            pltpu.VMEM((K,), jnp.int32),
            pltpu.VMEM((K,), jnp.int32),
            pltpu.SemaphoreType.DMA,
        ],
        compiler_params=pltpu.CompilerParams(use_tc_tiling_on_sc=False),
    )
    def sc_edge(g_hbm, ad_hbm, esrc_hbm, edst_hbm, z_hbm, u_hbm, u_sp, rows,
                adb, sidx, didx, sem):
        c = lax.axis_index("c")
        s = lax.axis_index("s")
        wid = c * NS + s
        pltpu.sync_copy(z_hbm.at[pl.ds(s * RPS, RPS)],
                        u_sp.at[pl.ds(s * RPS, RPS)])
        plsc.subcore_barrier()

        ebase = wid * EPW

        def chunk(ci, carry):
            eb = ebase + ci * K
            pltpu.sync_copy(esrc_hbm.at[pl.ds(eb, K)], sidx)
            pltpu.sync_copy(edst_hbm.at[pl.ds(eb, K)], didx)
            pltpu.async_copy(g_hbm.at[sidx], rows, sem).wait()
            pltpu.async_copy(ad_hbm.at[didx], adb, sem).wait()

            def edge(j, carry2):
                z = rows[j, pl.ds(128, 16)] + adb[j]
                z = jnp.where(z > 0.0, z, 0.2 * z)
                p = jnp.exp(z)
                rows[j, pl.ds(128, 16)] = p
                for b in range(8):
                    idx = jnp.full((16, 1), lane_for_block[b], jnp.int32)
                    pv = lax.gather(
                        p, idx,
                        lax.GatherDimensionNumbers(
                            offset_dims=(), collapsed_slice_dims=(0,),
                            start_index_map=(0,)),
                        slice_sizes=(1,),
                        mode=lax.GatherScatterMode.PROMISE_IN_BOUNDS)
                    rows[j, pl.ds(b * 16, 16)] = rows[j, pl.ds(b * 16, 16)] * pv
                return carry2

            lax.fori_loop(0, K, edge, 0)
            pltpu.sync_copy(rows, u_sp.at[didx], add=True)
            return carry

        lax.fori_loop(0, NCHUNK, chunk, 0)
        plsc.subcore_barrier()
        pltpu.sync_copy(u_sp.at[pl.ds(s * RPS, RPS)],
                        u_hbm.at[c, pl.ds(s * RPS, RPS)])

    return sc_edge


_sc_edge_l0 = _make_sc_edge(tuple(range(8)))
_sc_edge_l1 = _make_sc_edge((0,) * 8)


def kernel(x, edge_attr, edge_index, W0, att_src0, att_dst0, b0,
           W1, att_src1, att_dst1, b1):
    del edge_attr
    as0 = att_src0.reshape(1, 128)
    ad0 = att_dst0.reshape(1, 128)
    as1 = att_src1.reshape(1, 128)
    ad1 = att_dst1.reshape(1, 128)
    b0f = b0.reshape(1, 128)
    b1f = b1.reshape(1, 128)
    zeros = jnp.zeros((NPAD, FW), _f32)

    g0, adr0 = _tc_prep(x, W0, as0, ad0, 8)
    esrc = edge_index[0]
    edst = edge_index[1]
    u0 = _sc_edge_l0(g0, adr0, esrc, edst, zeros)
    g1, adr1 = _tc_mid(u0[0], u0[1], b0f, W1, as1, ad1)
    u1 = _sc_edge_l1(g1, adr1, esrc, edst, zeros)
    return _tc_final(u1[0], u1[1], b1f)


# speedup vs baseline: 68.3079x; 2.0089x over previous
import functools

import jax
import jax.numpy as jnp
from jax import lax
from jax.experimental import pallas as pl
from jax.experimental.pallas import tpu as pltpu
from jax.experimental.pallas import tpu_sc as plsc

N = 10000
E = 320000
ROWS = 256
NBLK = (N + ROWS - 1) // ROWS
NC, NS = 2, 16
NW = NC * NS
EPW = E // NW
K = 80
NCHUNK = EPW // K
NPAD = N
RPS = NPAD // NS
FW = 144

_f32 = jnp.float32


def _prep_math(x, w, att_s, att_d, heads):
    h = jnp.dot(x, w, preferred_element_type=_f32)
    ch = 128 // heads
    a_s = (h * att_s).reshape(ROWS, heads, ch).sum(-1)
    a_d = (h * att_d).reshape(ROWS, heads, ch).sum(-1)
    pad = jnp.zeros((ROWS, 16 - heads), _f32)
    g = jnp.concatenate([h, a_s, pad], axis=1)
    ad = jnp.concatenate([a_d, pad], axis=1)
    return g, ad


def _tc_prep_body(heads, x_ref, w_ref, as_ref, ad_ref, g_ref, adout_ref):
    g, ad = _prep_math(x_ref[...], w_ref[...], as_ref[...], ad_ref[...], heads)
    g_ref[...] = g
    adout_ref[...] = ad


def _tc_prep(x, w, att_s, att_d, heads):
    return pl.pallas_call(
        functools.partial(_tc_prep_body, heads),
        grid=(NBLK,),
        in_specs=[
            pl.BlockSpec((ROWS, 128), lambda i: (i, 0)),
            pl.BlockSpec((128, 128), lambda i: (0, 0)),
            pl.BlockSpec((1, 128), lambda i: (0, 0)),
            pl.BlockSpec((1, 128), lambda i: (0, 0)),
        ],
        out_specs=[
            pl.BlockSpec((ROWS, FW), lambda i: (i, 0)),
            pl.BlockSpec((ROWS, 16), lambda i: (i, 0)),
        ],
        out_shape=[
            jax.ShapeDtypeStruct((NPAD, FW), _f32),
            jax.ShapeDtypeStruct((NPAD, 16), _f32),
        ],
    )(x, w, att_s, att_d)


def _combine_math(u0, u1, b, heads):
    u = u0 + u1
    s = u[:, 128:128 + heads]
    o = u[:, :128].reshape(ROWS, heads, 128 // heads)
    o = o / (s + 1e-16)[:, :, None]
    return o.reshape(ROWS, 128) + b


def _tc_mid_body(u0_ref, u1_ref, b_ref, w_ref, as_ref, ad_ref, g_ref, adout_ref):
    x1 = _combine_math(u0_ref[...], u1_ref[...], b_ref[...], 8)
    x1 = jnp.where(x1 > 0.0, x1, jnp.exp(x1) - 1.0)
    g, ad = _prep_math(x1, w_ref[...], as_ref[...], ad_ref[...], 1)
    g_ref[...] = g
    adout_ref[...] = ad


def _tc_mid(u0, u1, b, w, att_s, att_d):
    return pl.pallas_call(
        _tc_mid_body,
        grid=(NBLK,),
        in_specs=[
            pl.BlockSpec((ROWS, FW), lambda i: (i, 0)),
            pl.BlockSpec((ROWS, FW), lambda i: (i, 0)),
            pl.BlockSpec((1, 128), lambda i: (0, 0)),
            pl.BlockSpec((128, 128), lambda i: (0, 0)),
            pl.BlockSpec((1, 128), lambda i: (0, 0)),
            pl.BlockSpec((1, 128), lambda i: (0, 0)),
        ],
        out_specs=[
            pl.BlockSpec((ROWS, FW), lambda i: (i, 0)),
            pl.BlockSpec((ROWS, 16), lambda i: (i, 0)),
        ],
        out_shape=[
            jax.ShapeDtypeStruct((NPAD, FW), _f32),
            jax.ShapeDtypeStruct((NPAD, 16), _f32),
        ],
    )(u0, u1, b, w, att_s, att_d)


def _tc_final_body(u0_ref, u1_ref, b_ref, out_ref):
    out_ref[...] = _combine_math(u0_ref[...], u1_ref[...], b_ref[...], 1)


def _tc_final(u0, u1, b):
    return pl.pallas_call(
        _tc_final_body,
        grid=(NBLK,),
        in_specs=[
            pl.BlockSpec((ROWS, FW), lambda i: (i, 0)),
            pl.BlockSpec((ROWS, FW), lambda i: (i, 0)),
            pl.BlockSpec((1, 128), lambda i: (0, 0)),
        ],
        out_specs=pl.BlockSpec((ROWS, 128), lambda i: (i, 0)),
        out_shape=jax.ShapeDtypeStruct((N, 128), _f32),
    )(u0, u1, b)


def _splat(p, lane):
    idx = jnp.full((16, 1), lane, jnp.int32)
    return lax.gather(
        p, idx,
        lax.GatherDimensionNumbers(
            offset_dims=(), collapsed_slice_dims=(0,), start_index_map=(0,)),
        slice_sizes=(1,),
        mode=lax.GatherScatterMode.PROMISE_IN_BOUNDS)


def _make_sc_edge(lane_for_block):
    mesh = plsc.VectorSubcoreMesh(core_axis_name="c", subcore_axis_name="s")
    NBUF = 3
    NIB = 4
    CYCLE = 12

    @functools.partial(
        pl.kernel,
        out_type=jax.ShapeDtypeStruct((NC, NPAD, FW), _f32),
        mesh=mesh,
        scratch_types=[
            pltpu.VMEM_SHARED((NPAD, FW), _f32),
            [pltpu.VMEM((K, FW), _f32) for _ in range(NBUF)],
            [pltpu.VMEM((K, 16), _f32) for _ in range(NBUF)],
            [pltpu.VMEM((K,), jnp.int32) for _ in range(NIB)],
            [pltpu.VMEM((K,), jnp.int32) for _ in range(NIB)],
            [pltpu.SemaphoreType.DMA for _ in range(NIB)],
            [pltpu.SemaphoreType.DMA for _ in range(NBUF)],
            [pltpu.SemaphoreType.DMA for _ in range(NBUF)],
        ],
        compiler_params=pltpu.CompilerParams(use_tc_tiling_on_sc=False),
    )
    def sc_edge(g_hbm, ad_hbm, esrc_hbm, edst_hbm, z_hbm, u_hbm, u_sp,
                rows, adb, sidx, didx, semi, semg, sems):
        c_ax = lax.axis_index("c")
        s_ax = lax.axis_index("s")
        wid = c_ax * NS + s_ax
        pltpu.sync_copy(z_hbm.at[pl.ds(s_ax * RPS, RPS)],
                        u_sp.at[pl.ds(s_ax * RPS, RPS)])
        plsc.subcore_barrier()

        def fire_idx(bi, ci):
            pltpu.async_copy(esrc_hbm.at[wid, ci], sidx[bi], semi[bi])
            pltpu.async_copy(edst_hbm.at[wid, ci], didx[bi], semi[bi])

        def wait_idx(bi, ci):
            pltpu.make_async_copy(esrc_hbm.at[wid, ci], sidx[bi],
                                  semi[bi]).wait()
            pltpu.make_async_copy(edst_hbm.at[wid, ci], didx[bi],
                                  semi[bi]).wait()

        def fire_gathers(b, bi):
            pltpu.async_copy(g_hbm.at[sidx[bi]], rows[b], semg[b])
            pltpu.async_copy(ad_hbm.at[didx[bi]], adb[b], semg[b])

        def wait_gathers(b, bi):
            pltpu.make_async_copy(g_hbm.at[sidx[bi]], rows[b], semg[b]).wait()
            pltpu.make_async_copy(ad_hbm.at[didx[bi]], adb[b], semg[b]).wait()

        def compute(b):
            r = rows[b]
            a = adb[b]

            def edge(j, carry2):
                z = r[j, pl.ds(128, 16)] + a[j]
                z = jnp.where(z > 0.0, z, 0.2 * z)
                p = jnp.exp(z)
                r[j, pl.ds(128, 16)] = p
                for blk in range(8):
                    pv = _splat(p, lane_for_block[blk])
                    r[j, pl.ds(blk * 16, 16)] = r[j, pl.ds(blk * 16, 16)] * pv
                return carry2

            lax.fori_loop(0, K, edge, 0, unroll=2)

        def fire_scatter(b, bi):
            pltpu.async_copy(rows[b], u_sp.at[didx[bi]], sems[b], add=True)

        def wait_scatter(b, bi):
            pltpu.make_async_copy(rows[b], u_sp.at[didx[bi]], sems[b]).wait()

        def step(c, first=False):
            b, bi = c % NBUF, c % NIB
            wait_gathers(b, bi)
            compute(b)
            fire_scatter(b, bi)
            if not first:
                pb = (c - 1) % NBUF
                wait_scatter(pb, (c - 1) % NIB)
            if c + 2 < NCHUNK:
                nb, nbi = (c + 2) % NBUF, (c + 2) % NIB
                wait_idx(nbi, c + 2)
                fire_gathers(nb, nbi)
            if c + 3 < NCHUNK:
                fire_idx((c + 3) % NIB, c + 3)

        for ci in range(3):
            fire_idx(ci % NIB, ci)
        for ci in range(2):
            wait_idx(ci % NIB, ci)
            fire_gathers(ci % NBUF, ci % NIB)

        nfull = (NCHUNK // CYCLE) * CYCLE

        def round12(g, carry):
            base = g * CYCLE
            for o in range(CYCLE):
                b, bi = o % NBUF, o % NIB
                cc = base + o
                wait_gathers(b, bi)
                compute(b)
                fire_scatter(b, bi)
                wait_scatter((o - 1) % NBUF, (o - 1) % NIB)
                wait_idx((o + 2) % NIB, cc + 2)
                fire_gathers((o + 2) % NBUF, (o + 2) % NIB)
                fire_idx((o + 3) % NIB, cc + 3)
            return carry

        for cc in range(CYCLE):
            step(cc, first=(cc == 0))
        lax.fori_loop(1, nfull // CYCLE, round12, 0)
        for cc in range(nfull, NCHUNK):
            step(cc)
        wait_scatter((NCHUNK - 1) % NBUF, (NCHUNK - 1) % NIB)
        plsc.subcore_barrier()
        pltpu.sync_copy(u_sp.at[pl.ds(s_ax * RPS, RPS)],
                        u_hbm.at[c_ax, pl.ds(s_ax * RPS, RPS)])

    return sc_edge


_sc_edge_l0 = _make_sc_edge(tuple(range(8)))
_sc_edge_l1 = _make_sc_edge((0,) * 8)


def kernel(x, edge_attr, edge_index, W0, att_src0, att_dst0, b0,
           W1, att_src1, att_dst1, b1):
    del edge_attr
    as0 = att_src0.reshape(1, 128)
    ad0 = att_dst0.reshape(1, 128)
    as1 = att_src1.reshape(1, 128)
    ad1 = att_dst1.reshape(1, 128)
    b0f = b0.reshape(1, 128)
    b1f = b1.reshape(1, 128)
    zeros = jnp.zeros((NPAD, FW), _f32)

    g0, adr0 = _tc_prep(x, W0, as0, ad0, 8)
    esrc = edge_index[0].reshape(NW, NCHUNK, K)
    edst = edge_index[1].reshape(NW, NCHUNK, K)
    u0 = _sc_edge_l0(g0, adr0, esrc, edst, zeros)
    g1, adr1 = _tc_mid(u0[0], u0[1], b0f, W1, as1, ad1)
    u1 = _sc_edge_l1(g1, adr1, esrc, edst, zeros)
    return _tc_final(u1[0], u1[1], b1f)
